# baseline (speedup 1.0000x reference)
import functools

import jax
import jax.numpy as jnp
from jax import lax
from jax.experimental import pallas as pl
from jax.experimental.pallas import tpu as pltpu
from jax.experimental.pallas import tpu_sc as plsc

_S = 32
_C = 16
_H = 32
_NW = 32


def _mm_body(x0_ref, x1_ref, x2_ref, x3_ref, w_ref, a128_ref,
             th_ref, tq_ref):
    w = w_ref[...]
    th2 = jnp.concatenate(
        [jnp.dot(x_ref[...], w, preferred_element_type=jnp.float32)
         for x_ref in (x0_ref, x1_ref, x2_ref, x3_ref)], axis=1)
    th_ref[...] = th2
    tq_ref[...] = jnp.dot(th2, a128_ref[...],
                          preferred_element_type=jnp.float32)


def _premul_table(feat_table, W, a12):
    n, d = feat_table.shape
    h = W.shape[1]
    rb = 1000
    nb = n // 4 // rb
    a128 = jnp.kron(jnp.eye(4, dtype=jnp.float32),
                    jnp.pad(a12, ((0, 0), (0, h - 2))))
    xspec = [pl.BlockSpec((rb, d), (lambda j: (lambda i: (i + nb * j, 0)))(j))
             for j in range(4)]
    return pl.pallas_call(
        _mm_body,
        grid=(nb,),
        in_specs=xspec + [
            pl.BlockSpec((d, h), lambda i: (0, 0)),
            pl.BlockSpec((128, 128), lambda i: (0, 0)),
        ],
        out_specs=[
            pl.BlockSpec((rb, 128), lambda i: (i, 0)),
            pl.BlockSpec((rb, 128), lambda i: (i, 0)),
        ],
        out_shape=[
            jax.ShapeDtypeStruct((n // 4, 128), jnp.float32),
            jax.ShapeDtypeStruct((n // 4, 128), jnp.float32),
        ],
    )(feat_table, feat_table, feat_table, feat_table, W, a128)


def _sc_attention(table, tq, nbr3d, nbrq3d, nodeq3d, vslot, n_chunks):
    mesh = plsc.VectorSubcoreMesh(core_axis_name="c", subcore_axis_name="s")
    nc = mesh.num_cores
    b_pad = _NW * n_chunks * _C
    nir = (_C * _S) // 128

    @functools.partial(
        pl.kernel,
        out_type=(
            jax.ShapeDtypeStruct((b_pad, _S), jnp.float32),
            jax.ShapeDtypeStruct((_C, _S), jnp.float32),
        ),
        mesh=mesh,
        compiler_params=pltpu.CompilerParams(use_tc_tiling_on_sc=False),
        scratch_types=[
            pltpu.VMEM((n_chunks + 2, nir, 128), jnp.int32),
            pltpu.VMEM((n_chunks + 2, nir, 128), jnp.int32),
            pltpu.VMEM((n_chunks + 2, 1, _C), jnp.int32),
            pltpu.VMEM((2, _C * _S, _H), jnp.float32),
            pltpu.VMEM((2, _C * _S), jnp.float32),
            pltpu.VMEM((2, _C), jnp.float32),
            pltpu.VMEM((2, 16), jnp.float32),
            pltpu.VMEM((2, _S, 16), jnp.float32),
            pltpu.VMEM((2, _C, _S), jnp.float32),
            pltpu.SemaphoreType.DMA,
            pltpu.SemaphoreType.DMA,
            pltpu.SemaphoreType.DMA,
            pltpu.SemaphoreType.DMA,
        ],
    )
    def k(table_hbm, tq_hbm, nbr_hbm, nbrq_hbm, nodeq_hbm, v_hbm,
          out_hbm, dummy_hbm,
          idx_v, idxq_v, cdx_v, rows_v, tn_v, tc_v, v_v, coef_v, out_v,
          g0, g1, o0, o1):
        cid = lax.axis_index("c")
        sid = lax.axis_index("s")
        wid = sid * nc + cid
        gsem = (g0, g1)
        osem = (o0, o1)

        pltpu.sync_copy(v_hbm, v_v)
        v0 = v_v[0, :]
        v1 = v_v[1, :]
        vs = [v0[i] for i in range(16)] + [v1[i] for i in range(16)]

        pltpu.sync_copy(nbr_hbm.at[pl.ds(wid * n_chunks, n_chunks + 2)], idx_v)
        pltpu.sync_copy(nbrq_hbm.at[pl.ds(wid * n_chunks, n_chunks + 2)],
                        idxq_v)
        pltpu.sync_copy(nodeq_hbm.at[pl.ds(wid * n_chunks, n_chunks + 2)],
                        cdx_v)

        def gathers(c, slot):
            ds = []
            for j in range(nir):
                ds.append(pltpu.async_copy(
                    table_hbm.at[idx_v.at[c, j]],
                    rows_v.at[slot].at[pl.ds(j * 128, 128)], gsem[slot]))
                ds.append(pltpu.async_copy(
                    tq_hbm.at[idxq_v.at[c, j]],
                    tn_v.at[slot].at[pl.ds(j * 128, 128)], gsem[slot]))
            ds.append(pltpu.async_copy(
                tq_hbm.at[cdx_v.at[c, 0]], tc_v.at[slot], gsem[slot]))
            return ds

        def wait_gathers(c, slot):
            for j in range(nir):
                pltpu.make_async_copy(
                    table_hbm.at[idx_v.at[c, j]],
                    rows_v.at[slot].at[pl.ds(j * 128, 128)], gsem[slot]).wait()
                pltpu.make_async_copy(
                    tq_hbm.at[idxq_v.at[c, j]],
                    tn_v.at[slot].at[pl.ds(j * 128, 128)], gsem[slot]).wait()
            pltpu.make_async_copy(
                tq_hbm.at[cdx_v.at[c, 0]], tc_v.at[slot], gsem[slot]).wait()

        def wait_out(slot):
            pltpu.make_async_copy(out_v.at[slot], dummy_hbm, osem[slot]).wait()

        def compute(c, slot):
            rows = rows_v.at[slot]
            tn = tn_v.at[slot]
            coef = coef_v.at[slot]
            out = out_v.at[slot]
            t2row = tc_v[slot, :]
            rs = jnp.zeros((16,), jnp.float32)
            for s in range(_S):
                t1s = tn[pl.ds(s * 16, 16)]
                lg = t1s + t2row
                lk = jnp.where(lg >= 0, lg, 0.2 * lg)
                e = jnp.exp(-lk)
                coef[s, :] = e
                rs = rs + e
            inv = jnp.where(rs > 0.0, 1.0 / rs,
                            jnp.zeros((16,), jnp.float32))
            for s in range(_S):
                coef[s, :] = coef[s, :] * (inv * vs[s])

            def body(s, accs):
                cvec = coef[s, :]
                base = s * 16
                new = []
                for b in range(_C):
                    cb = cvec[b]
                    new.append(accs[2 * b]
                               + cb * rows[base + b, pl.ds(0, 16)])
                    new.append(accs[2 * b + 1]
                               + cb * rows[base + b, pl.ds(16, 16)])
                return tuple(new)

            zeros = jnp.zeros((16,), jnp.float32)
            accs = lax.fori_loop(0, _S, body, (zeros,) * (2 * _C))
            for b in range(_C):
                out[b, pl.ds(0, 16)] = jnp.maximum(accs[2 * b], 0.0)
                out[b, pl.ds(16, 16)] = jnp.maximum(accs[2 * b + 1], 0.0)
            base_b = (wid * n_chunks + c) * _C
            pltpu.async_copy(out_v.at[slot],
                             out_hbm.at[pl.ds(base_b, _C)], osem[slot])

        gathers(0, 0)
        for slot in range(2):
            pltpu.async_copy(out_v.at[slot], dummy_hbm, osem[slot])

        def pair(kk, carry):
            c = 2 * kk
            gathers(c + 1, 1)
            wait_out(0)
            wait_gathers(c, 0)
            compute(c, 0)
            gathers(c + 2, 0)
            wait_out(1)
            wait_gathers(c + 1, 1)
            compute(c + 1, 1)
            return carry

        lax.fori_loop(0, n_chunks // 2, pair, 0)
        wait_gathers(n_chunks, 0)
        for slot in range(2):
            wait_out(slot)

    return k(table, tq, nbr3d, nbrq3d, nodeq3d, vslot)


def kernel(nodes, neighbors, feat_table, W, a, U):
    b, s = neighbors.shape
    h = W.shape[1]
    fs = U.shape[0]
    a_ctr, a_nbr = a[0, :h], a[0, h:]
    colsum = jnp.sum(U, axis=0)
    v = jnp.sum(U * colsum[None, :], axis=1)
    vslot = v[1:1 + s].reshape(2, 16)
    a12 = jnp.stack([a_nbr, a_ctr], axis=1)
    th3, tq3 = _premul_table(feat_table, W, a12)
    table = th3.reshape(-1, h)
    tq = tq3.reshape(-1)

    n_chunks = -(-b // (_NW * _C))
    b_pad = _NW * _C * n_chunks
    nodes_p = jnp.pad(nodes, (0, b_pad + 2 * _C - b))
    nbr_p = jnp.pad(neighbors, ((0, b_pad + 2 * _C - b), (0, 0)))
    q = feat_table.shape[0] // 4
    nbr_lo, nbr_hi = nbr_p % q, nbr_p // q
    nbr_rows = nbr_lo * 4 + nbr_hi

    def to3d(x):
        return (x.reshape(-1, _C, _S)
                .transpose(0, 2, 1)
                .reshape(-1, (_C * _S) // 128, 128))

    nbr3d = to3d(nbr_rows * 0)
    nbrq3d = to3d((nbr_lo * 128 + nbr_hi * 32) * 0)
    nodeq3d = ((nodes_p % q) * 128 + (nodes_p // q) * 32 + 1
               ).reshape(-1, 1, _C)
    out, _ = _sc_attention(table, tq, nbr3d, nbrq3d, nodeq3d, vslot,
                           n_chunks)
    return out[:b]


# speedup vs baseline: 21.9231x; 21.9231x over previous
import functools

import jax
import jax.numpy as jnp
from jax import lax
from jax.experimental import pallas as pl
from jax.experimental.pallas import tpu as pltpu
from jax.experimental.pallas import tpu_sc as plsc

_S = 32
_C = 16
_H = 32
_NW = 32


def _mm_body(x0_ref, x1_ref, x2_ref, x3_ref, w_ref, a128_ref,
             th_ref, tq_ref):
    w = w_ref[...]
    th2 = jnp.concatenate(
        [jnp.dot(x_ref[...], w, preferred_element_type=jnp.float32)
         for x_ref in (x0_ref, x1_ref, x2_ref, x3_ref)], axis=1)
    th_ref[...] = th2
    tq_ref[...] = jnp.dot(th2, a128_ref[...],
                          preferred_element_type=jnp.float32)


def _premul_table(feat_table, W, a12):
    n, d = feat_table.shape
    h = W.shape[1]
    rb = 1000
    nb = n // 4 // rb
    a128 = jnp.kron(jnp.eye(4, dtype=jnp.float32),
                    jnp.pad(a12, ((0, 0), (0, h - 2))))
    xspec = [pl.BlockSpec((rb, d), (lambda j: (lambda i: (i + nb * j, 0)))(j))
             for j in range(4)]
    return pl.pallas_call(
        _mm_body,
        grid=(nb,),
        in_specs=xspec + [
            pl.BlockSpec((d, h), lambda i: (0, 0)),
            pl.BlockSpec((128, 128), lambda i: (0, 0)),
        ],
        out_specs=[
            pl.BlockSpec((rb, 128), lambda i: (i, 0)),
            pl.BlockSpec((rb, 128), lambda i: (i, 0)),
        ],
        out_shape=[
            jax.ShapeDtypeStruct((n // 4, 128), jnp.float32),
            jax.ShapeDtypeStruct((n // 4, 128), jnp.float32),
        ],
    )(feat_table, feat_table, feat_table, feat_table, W, a128)


def _sc_attention(table, tq, nbr3d, nbrq3d, nodeq3d, vslot, n_chunks):
    mesh = plsc.VectorSubcoreMesh(core_axis_name="c", subcore_axis_name="s")
    nc = mesh.num_cores
    b_pad = _NW * n_chunks * _C
    nir = (_C * _S) // 128

    @functools.partial(
        pl.kernel,
        out_type=(
            jax.ShapeDtypeStruct((b_pad, _S), jnp.float32),
            jax.ShapeDtypeStruct((_C, _S), jnp.float32),
        ),
        mesh=mesh,
        compiler_params=pltpu.CompilerParams(use_tc_tiling_on_sc=False),
        scratch_types=[
            pltpu.VMEM((n_chunks + 2, nir, 128), jnp.int32),
            pltpu.VMEM((n_chunks + 2, nir, 128), jnp.int32),
            pltpu.VMEM((n_chunks + 2, 1, _C), jnp.int32),
            pltpu.VMEM((2, _C * _S, _H), jnp.float32),
            pltpu.VMEM((2, _C * _S), jnp.float32),
            pltpu.VMEM((2, _C), jnp.float32),
            pltpu.VMEM((2, 16), jnp.float32),
            pltpu.VMEM((2, _S, 16), jnp.float32),
            pltpu.VMEM((2, _C, _S), jnp.float32),
            pltpu.SemaphoreType.DMA,
            pltpu.SemaphoreType.DMA,
            pltpu.SemaphoreType.DMA,
            pltpu.SemaphoreType.DMA,
        ],
    )
    def k(table_hbm, tq_hbm, nbr_hbm, nbrq_hbm, nodeq_hbm, v_hbm,
          out_hbm, dummy_hbm,
          idx_v, idxq_v, cdx_v, rows_v, tn_v, tc_v, v_v, coef_v, out_v,
          g0, g1, o0, o1):
        cid = lax.axis_index("c")
        sid = lax.axis_index("s")
        wid = sid * nc + cid
        gsem = (g0, g1)
        osem = (o0, o1)

        pltpu.sync_copy(v_hbm, v_v)
        v0 = v_v[0, :]
        v1 = v_v[1, :]
        vs = [v0[i] for i in range(16)] + [v1[i] for i in range(16)]

        pltpu.sync_copy(nbr_hbm.at[pl.ds(wid * n_chunks, n_chunks + 2)], idx_v)
        pltpu.sync_copy(nbrq_hbm.at[pl.ds(wid * n_chunks, n_chunks + 2)],
                        idxq_v)
        pltpu.sync_copy(nodeq_hbm.at[pl.ds(wid * n_chunks, n_chunks + 2)],
                        cdx_v)

        def gathers(c, slot):
            ds = []
            for j in range(nir):
                ds.append(pltpu.async_copy(
                    table_hbm.at[idx_v.at[c, j]],
                    rows_v.at[slot].at[pl.ds(j * 128, 128)], gsem[slot]))
                ds.append(pltpu.async_copy(
                    tq_hbm.at[idxq_v.at[c, j]],
                    tn_v.at[slot].at[pl.ds(j * 128, 128)], gsem[slot]))
            ds.append(pltpu.async_copy(
                tq_hbm.at[cdx_v.at[c, 0]], tc_v.at[slot], gsem[slot]))
            return ds

        def wait_gathers(c, slot):
            for j in range(nir):
                pltpu.make_async_copy(
                    table_hbm.at[idx_v.at[c, j]],
                    rows_v.at[slot].at[pl.ds(j * 128, 128)], gsem[slot]).wait()
                pltpu.make_async_copy(
                    tq_hbm.at[idxq_v.at[c, j]],
                    tn_v.at[slot].at[pl.ds(j * 128, 128)], gsem[slot]).wait()
            pltpu.make_async_copy(
                tq_hbm.at[cdx_v.at[c, 0]], tc_v.at[slot], gsem[slot]).wait()

        def wait_out(slot):
            pltpu.make_async_copy(out_v.at[slot], dummy_hbm, osem[slot]).wait()

        def compute(c, slot):
            rows = rows_v.at[slot]
            tn = tn_v.at[slot]
            coef = coef_v.at[slot]
            out = out_v.at[slot]
            t2row = tc_v[slot, :]
            rs = jnp.zeros((16,), jnp.float32)
            for s in range(_S):
                t1s = tn[pl.ds(s * 16, 16)]
                lg = t1s + t2row
                lk = jnp.where(lg >= 0, lg, 0.2 * lg)
                e = jnp.exp(-lk)
                coef[s, :] = e
                rs = rs + e
            inv = jnp.where(rs > 0.0, 1.0 / rs,
                            jnp.zeros((16,), jnp.float32))
            for s in range(_S):
                coef[s, :] = coef[s, :] * (inv * vs[s])

            def body(s, accs):
                cvec = coef[s, :]
                base = s * 16
                new = []
                for b in range(_C):
                    cb = cvec[b]
                    new.append(accs[2 * b]
                               + cb * rows[base + b, pl.ds(0, 16)])
                    new.append(accs[2 * b + 1]
                               + cb * rows[base + b, pl.ds(16, 16)])
                return tuple(new)

            zeros = jnp.zeros((16,), jnp.float32)
            accs = lax.fori_loop(0, _S, body, (zeros,) * (2 * _C))
            for b in range(_C):
                out[b, pl.ds(0, 16)] = jnp.maximum(accs[2 * b], 0.0)
                out[b, pl.ds(16, 16)] = jnp.maximum(accs[2 * b + 1], 0.0)
            base_b = (wid * n_chunks + c) * _C
            pltpu.async_copy(out_v.at[slot],
                             out_hbm.at[pl.ds(base_b, _C)], osem[slot])

        gathers(0, 0)
        for slot in range(2):
            pltpu.async_copy(out_v.at[slot], dummy_hbm, osem[slot])

        def pair(kk, carry):
            c = 2 * kk
            gathers(c + 1, 1)
            wait_out(0)
            wait_gathers(c, 0)
            compute(c, 0)
            gathers(c + 2, 0)
            wait_out(1)
            wait_gathers(c + 1, 1)
            compute(c + 1, 1)
            return carry

        lax.fori_loop(0, n_chunks // 2, pair, 0)
        wait_gathers(n_chunks, 0)
        for slot in range(2):
            wait_out(slot)

    return k(table, tq, nbr3d, nbrq3d, nodeq3d, vslot)


def kernel(nodes, neighbors, feat_table, W, a, U):
    b, s = neighbors.shape
    h = W.shape[1]
    fs = U.shape[0]
    a_ctr, a_nbr = a[0, :h], a[0, h:]
    colsum = jnp.sum(U, axis=0)
    v = jnp.sum(U * colsum[None, :], axis=1)
    vslot = v[1:1 + s].reshape(2, 16)
    a12 = jnp.stack([a_nbr, a_ctr], axis=1)
    th3, tq3 = _premul_table(feat_table, W, a12)
    table = th3.reshape(-1, h)
    tq = tq3.reshape(-1)

    n_chunks = -(-b // (_NW * _C))
    b_pad = _NW * _C * n_chunks
    n_nodes = feat_table.shape[0]
    pad_n = b_pad + 2 * _C - b
    nodes_p = jnp.concatenate(
        [nodes, (jnp.arange(pad_n, dtype=jnp.int32) * 97) % n_nodes])
    nbr_fill = ((jnp.arange(pad_n * _S, dtype=jnp.int32) * 89) % n_nodes
                ).reshape(pad_n, _S)
    nbr_p = jnp.concatenate([neighbors, nbr_fill])
    q = feat_table.shape[0] // 4
    nbr_lo, nbr_hi = nbr_p % q, nbr_p // q
    nbr_rows = nbr_lo * 4 + nbr_hi

    def to3d(x):
        return (x.reshape(-1, _C, _S)
                .transpose(0, 2, 1)
                .reshape(-1, (_C * _S) // 128, 128))

    nbr3d = to3d(nbr_rows)
    nbrq3d = to3d(nbr_lo * 128 + nbr_hi * 32)
    nodeq3d = ((nodes_p % q) * 128 + (nodes_p // q) * 32 + 1
               ).reshape(-1, 1, _C)
    out, _ = _sc_attention(table, tq, nbr3d, nbrq3d, nodeq3d, vslot,
                           n_chunks)
    return out[:b]


# speedup vs baseline: 23.5928x; 1.0762x over previous
import functools

import jax
import jax.numpy as jnp
from jax import lax
from jax.experimental import pallas as pl
from jax.experimental.pallas import tpu as pltpu
from jax.experimental.pallas import tpu_sc as plsc

_S = 32
_C = 16
_H = 32
_NW = 32


def _mm_body(x0_ref, x1_ref, x2_ref, x3_ref, w_ref, a128_ref,
             th_ref, tq_ref):
    w = w_ref[...]
    th2 = jnp.concatenate(
        [jnp.dot(x_ref[...], w, preferred_element_type=jnp.float32)
         for x_ref in (x0_ref, x1_ref, x2_ref, x3_ref)], axis=1)
    th_ref[...] = th2
    tq_ref[...] = jnp.dot(th2, a128_ref[...],
                          preferred_element_type=jnp.float32)


def _premul_table(feat_table, W, a12):
    n, d = feat_table.shape
    h = W.shape[1]
    rb = 1000
    nb = n // 4 // rb
    a128 = jnp.kron(jnp.eye(4, dtype=jnp.float32),
                    jnp.pad(a12, ((0, 0), (0, h - 2))))
    xspec = [pl.BlockSpec((rb, d), (lambda j: (lambda i: (i + nb * j, 0)))(j))
             for j in range(4)]
    return pl.pallas_call(
        _mm_body,
        grid=(nb,),
        in_specs=xspec + [
            pl.BlockSpec((d, h), lambda i: (0, 0)),
            pl.BlockSpec((128, 128), lambda i: (0, 0)),
        ],
        out_specs=[
            pl.BlockSpec((rb, 128), lambda i: (i, 0)),
            pl.BlockSpec((rb, 128), lambda i: (i, 0)),
        ],
        out_shape=[
            jax.ShapeDtypeStruct((n // 4, 128), jnp.float32),
            jax.ShapeDtypeStruct((n // 4, 128), jnp.float32),
        ],
    )(feat_table, feat_table, feat_table, feat_table, W, a128)


def _sc_attention(table, tq, nbr3d, nodeq3d, vslot, n_chunks):
    mesh = plsc.VectorSubcoreMesh(core_axis_name="c", subcore_axis_name="s")
    nc = mesh.num_cores
    b_pad = _NW * n_chunks * _C
    nir = (_C * _S) // 128

    @functools.partial(
        pl.kernel,
        out_type=(
            jax.ShapeDtypeStruct((b_pad, _S), jnp.float32),
            jax.ShapeDtypeStruct((_C, _S), jnp.float32),
        ),
        mesh=mesh,
        compiler_params=pltpu.CompilerParams(use_tc_tiling_on_sc=False),
        scratch_types=[
            pltpu.VMEM((n_chunks + 2, nir, 128), jnp.int32),
            pltpu.VMEM((2, nir, 128), jnp.int32),
            pltpu.VMEM((n_chunks + 2, 1, _C), jnp.int32),
            pltpu.VMEM((2, _C * _S, _H), jnp.float32),
            pltpu.VMEM((2, _C * _S), jnp.float32),
            pltpu.VMEM((2, _C), jnp.float32),
            pltpu.VMEM((2, 16), jnp.float32),
            pltpu.VMEM((2, _S, 16), jnp.float32),
            pltpu.VMEM((2, _C, _S), jnp.float32),
            pltpu.SemaphoreType.DMA,
            pltpu.SemaphoreType.DMA,
            pltpu.SemaphoreType.DMA,
            pltpu.SemaphoreType.DMA,
        ],
    )
    def k(table_hbm, tq_hbm, nbr_hbm, nodeq_hbm, v_hbm,
          out_hbm, dummy_hbm,
          idx_v, idxq_v, cdx_v, rows_v, tn_v, tc_v, v_v, coef_v, out_v,
          g0, g1, o0, o1):
        cid = lax.axis_index("c")
        sid = lax.axis_index("s")
        wid = sid * nc + cid
        gsem = (g0, g1)
        osem = (o0, o1)

        pltpu.sync_copy(v_hbm, v_v)
        v0 = v_v[0, :]
        v1 = v_v[1, :]
        vs = [v0[i] for i in range(16)] + [v1[i] for i in range(16)]

        pltpu.sync_copy(nbr_hbm.at[pl.ds(wid * n_chunks, n_chunks + 2)], idx_v)
        pltpu.sync_copy(nodeq_hbm.at[pl.ds(wid * n_chunks, n_chunks + 2)],
                        cdx_v)

        def gathers(c, slot):
            for j in range(nir):
                for kk in range(8):
                    sl = pl.ds(kk * 16, 16)
                    idxq_v[slot, j, sl] = idx_v[c, j, sl] * 32
            ds = []
            for j in range(nir):
                ds.append(pltpu.async_copy(
                    table_hbm.at[idx_v.at[c, j]],
                    rows_v.at[slot].at[pl.ds(j * 128, 128)], gsem[slot]))
                ds.append(pltpu.async_copy(
                    tq_hbm.at[idxq_v.at[slot, j]],
                    tn_v.at[slot].at[pl.ds(j * 128, 128)], gsem[slot]))
            ds.append(pltpu.async_copy(
                tq_hbm.at[cdx_v.at[c, 0]], tc_v.at[slot], gsem[slot]))
            return ds

        def wait_gathers(c, slot):
            for j in range(nir):
                pltpu.make_async_copy(
                    table_hbm.at[idx_v.at[c, j]],
                    rows_v.at[slot].at[pl.ds(j * 128, 128)], gsem[slot]).wait()
                pltpu.make_async_copy(
                    tq_hbm.at[idxq_v.at[slot, j]],
                    tn_v.at[slot].at[pl.ds(j * 128, 128)], gsem[slot]).wait()
            pltpu.make_async_copy(
                tq_hbm.at[cdx_v.at[c, 0]], tc_v.at[slot], gsem[slot]).wait()

        def wait_out(slot):
            pltpu.make_async_copy(out_v.at[slot], dummy_hbm, osem[slot]).wait()

        def compute(c, slot):
            rows = rows_v.at[slot]
            tn = tn_v.at[slot]
            coef = coef_v.at[slot]
            out = out_v.at[slot]
            t2row = tc_v[slot, :]
            rs = jnp.zeros((16,), jnp.float32)
            for s in range(_S):
                t1s = tn[pl.ds(s * 16, 16)]
                lg = t1s + t2row
                lk = jnp.where(lg >= 0, lg, 0.2 * lg)
                e = jnp.exp(-lk)
                coef[s, :] = e
                rs = rs + e
            inv = jnp.where(rs > 0.0, 1.0 / rs,
                            jnp.zeros((16,), jnp.float32))
            for s in range(_S):
                coef[s, :] = coef[s, :] * (inv * vs[s])

            def body(s, accs):
                cvec = coef[s, :]
                base = s * 16
                new = []
                for b in range(_C):
                    cb = cvec[b]
                    new.append(accs[2 * b]
                               + cb * rows[base + b, pl.ds(0, 16)])
                    new.append(accs[2 * b + 1]
                               + cb * rows[base + b, pl.ds(16, 16)])
                return tuple(new)

            zeros = jnp.zeros((16,), jnp.float32)
            accs = lax.fori_loop(0, _S, body, (zeros,) * (2 * _C))
            for b in range(_C):
                out[b, pl.ds(0, 16)] = jnp.maximum(accs[2 * b], 0.0)
                out[b, pl.ds(16, 16)] = jnp.maximum(accs[2 * b + 1], 0.0)
            base_b = (wid * n_chunks + c) * _C
            pltpu.async_copy(out_v.at[slot],
                             out_hbm.at[pl.ds(base_b, _C)], osem[slot])

        gathers(0, 0)
        for slot in range(2):
            pltpu.async_copy(out_v.at[slot], dummy_hbm, osem[slot])

        def pair(kk, carry):
            c = 2 * kk
            gathers(c + 1, 1)
            wait_out(0)
            wait_gathers(c, 0)
            compute(c, 0)
            gathers(c + 2, 0)
            wait_out(1)
            wait_gathers(c + 1, 1)
            compute(c + 1, 1)
            return carry

        lax.fori_loop(0, n_chunks // 2, pair, 0)
        wait_gathers(n_chunks, 0)
        for slot in range(2):
            wait_out(slot)

    return k(table, tq, nbr3d, nodeq3d, vslot)


def kernel(nodes, neighbors, feat_table, W, a, U):
    b, s = neighbors.shape
    h = W.shape[1]
    fs = U.shape[0]
    a_ctr, a_nbr = a[0, :h], a[0, h:]
    colsum = jnp.sum(U, axis=0)
    v = jnp.sum(U * colsum[None, :], axis=1)
    vslot = v[1:1 + s].reshape(2, 16)
    a12 = jnp.stack([a_nbr, a_ctr], axis=1)
    th3, tq3 = _premul_table(feat_table, W, a12)
    table = th3.reshape(-1, h)
    tq = tq3.reshape(-1)

    n_chunks = -(-b // (_NW * _C))
    b_pad = _NW * _C * n_chunks
    n_nodes = feat_table.shape[0]
    pad_n = b_pad + 2 * _C - b
    nodes_p = jnp.concatenate(
        [nodes, (jnp.arange(pad_n, dtype=jnp.int32) * 97) % n_nodes])
    nbr_fill = ((jnp.arange(pad_n * _S, dtype=jnp.int32) * 89) % n_nodes
                ).reshape(pad_n, _S)
    nbr_p = jnp.concatenate([neighbors, nbr_fill])
    q = feat_table.shape[0] // 4
    nbr_lo, nbr_hi = nbr_p % q, nbr_p // q
    nbr_rows = nbr_lo * 4 + nbr_hi

    def to3d(x):
        return (x.reshape(-1, _C, _S)
                .transpose(0, 2, 1)
                .reshape(-1, (_C * _S) // 128, 128))

    nbr3d = to3d(nbr_rows)
    nodeq3d = (((nodes_p % q) * 4 + nodes_p // q) * 32 + 1
               ).reshape(-1, 1, _C)
    out, _ = _sc_attention(table, tq, nbr3d, nodeq3d, vslot, n_chunks)
    return out[:b]


# speedup vs baseline: 24.3893x; 1.0338x over previous
import functools

import jax
import jax.numpy as jnp
from jax import lax
from jax.experimental import pallas as pl
from jax.experimental.pallas import tpu as pltpu
from jax.experimental.pallas import tpu_sc as plsc

_S = 32
_C = 16
_H = 32
_NW = 32


def _mm_body(x0_ref, x1_ref, x2_ref, x3_ref, w_ref, a128_ref,
             th_ref, tq_ref):
    w = w_ref[...]
    th2 = jnp.concatenate(
        [jnp.dot(x_ref[...], w, preferred_element_type=jnp.float32)
         for x_ref in (x0_ref, x1_ref, x2_ref, x3_ref)], axis=1)
    th_ref[...] = th2
    tq_ref[...] = jnp.dot(th2, a128_ref[...],
                          preferred_element_type=jnp.float32)


def _premul_table(feat_table, W, a12):
    n, d = feat_table.shape
    h = W.shape[1]
    rb = 1000
    nb = n // 4 // rb
    a128 = jnp.kron(jnp.eye(4, dtype=jnp.float32),
                    jnp.pad(a12, ((0, 0), (0, h - 2))))
    xspec = [pl.BlockSpec((rb, d), (lambda j: (lambda i: (i + nb * j, 0)))(j))
             for j in range(4)]
    return pl.pallas_call(
        _mm_body,
        grid=(nb,),
        in_specs=xspec + [
            pl.BlockSpec((d, h), lambda i: (0, 0)),
            pl.BlockSpec((128, 128), lambda i: (0, 0)),
        ],
        out_specs=[
            pl.BlockSpec((rb, 128), lambda i: (i, 0)),
            pl.BlockSpec((rb, 128), lambda i: (i, 0)),
        ],
        out_shape=[
            jax.ShapeDtypeStruct((n // 4, 128), jnp.float32),
            jax.ShapeDtypeStruct((n // 4, 128), jnp.float32),
        ],
    )(feat_table, feat_table, feat_table, feat_table, W, a128)


def _sc_attention(table, tq, nbr3d, nodeq3d, vslot, n_chunks, b_out):
    mesh = plsc.VectorSubcoreMesh(core_axis_name="c", subcore_axis_name="s")
    nc = mesh.num_cores
    nir = (_C * _S) // 128

    @functools.partial(
        pl.kernel,
        out_type=(
            jax.ShapeDtypeStruct((b_out, _S), jnp.float32),
            jax.ShapeDtypeStruct((_C, _S), jnp.float32),
        ),
        mesh=mesh,
        compiler_params=pltpu.CompilerParams(use_tc_tiling_on_sc=False),
        scratch_types=[
            pltpu.VMEM((n_chunks + 2, nir, 128), jnp.int32),
            pltpu.VMEM((2, nir, 128), jnp.int32),
            pltpu.VMEM((n_chunks + 2, 1, _C), jnp.int32),
            pltpu.VMEM((2, _C * _S, _H), jnp.float32),
            pltpu.VMEM((2, _C * _S), jnp.float32),
            pltpu.VMEM((2, _C), jnp.float32),
            pltpu.VMEM((2, 16), jnp.float32),
            pltpu.VMEM((2, _S, 16), jnp.float32),
            pltpu.VMEM((2, _C, _S), jnp.float32),
            pltpu.SemaphoreType.DMA,
            pltpu.SemaphoreType.DMA,
            pltpu.SemaphoreType.DMA,
            pltpu.SemaphoreType.DMA,
        ],
    )
    def k(table_hbm, tq_hbm, nbr_hbm, nodeq_hbm, v_hbm,
          out_hbm, dummy_hbm,
          idx_v, idxq_v, cdx_v, rows_v, tn_v, tc_v, v_v, coef_v, out_v,
          g0, g1, o0, o1):
        cid = lax.axis_index("c")
        sid = lax.axis_index("s")
        wid = sid * nc + cid
        gsem = (g0, g1)
        osem = (o0, o1)

        pltpu.sync_copy(v_hbm, v_v)
        v0 = v_v[0, :]
        v1 = v_v[1, :]
        vs = [v0[i] for i in range(16)] + [v1[i] for i in range(16)]

        pltpu.sync_copy(nbr_hbm.at[pl.ds(wid * n_chunks, n_chunks + 2)], idx_v)
        pltpu.sync_copy(nodeq_hbm.at[pl.ds(wid * n_chunks, n_chunks + 2)],
                        cdx_v)

        def gathers(c, slot):
            for j in range(nir):
                for kk in range(8):
                    sl = pl.ds(kk * 16, 16)
                    idxq_v[slot, j, sl] = idx_v[c, j, sl] * 32
            ds = []
            for j in range(nir):
                ds.append(pltpu.async_copy(
                    table_hbm.at[idx_v.at[c, j]],
                    rows_v.at[slot].at[pl.ds(j * 128, 128)], gsem[slot]))
                ds.append(pltpu.async_copy(
                    tq_hbm.at[idxq_v.at[slot, j]],
                    tn_v.at[slot].at[pl.ds(j * 128, 128)], gsem[slot]))
            ds.append(pltpu.async_copy(
                tq_hbm.at[cdx_v.at[c, 0]], tc_v.at[slot], gsem[slot]))
            return ds

        def wait_gathers(c, slot):
            for j in range(nir):
                pltpu.make_async_copy(
                    table_hbm.at[idx_v.at[c, j]],
                    rows_v.at[slot].at[pl.ds(j * 128, 128)], gsem[slot]).wait()
                pltpu.make_async_copy(
                    tq_hbm.at[idxq_v.at[slot, j]],
                    tn_v.at[slot].at[pl.ds(j * 128, 128)], gsem[slot]).wait()
            pltpu.make_async_copy(
                tq_hbm.at[cdx_v.at[c, 0]], tc_v.at[slot], gsem[slot]).wait()

        def wait_out(slot):
            pltpu.make_async_copy(out_v.at[slot], dummy_hbm, osem[slot]).wait()

        def compute(c, slot):
            rows = rows_v.at[slot]
            tn = tn_v.at[slot]
            coef = coef_v.at[slot]
            out = out_v.at[slot]
            t2row = tc_v[slot, :]
            rs = jnp.zeros((16,), jnp.float32)
            for s in range(_S):
                t1s = tn[pl.ds(s * 16, 16)]
                lg = t1s + t2row
                lk = jnp.where(lg >= 0, lg, 0.2 * lg)
                e = jnp.exp(-lk)
                coef[s, :] = e
                rs = rs + e
            inv = jnp.where(rs > 0.0, 1.0 / rs,
                            jnp.zeros((16,), jnp.float32))
            for s in range(_S):
                coef[s, :] = coef[s, :] * (inv * vs[s])

            def body(s, accs):
                cvec = coef[s, :]
                base = s * 16
                new = []
                for b in range(_C):
                    cb = cvec[b]
                    new.append(accs[2 * b]
                               + cb * rows[base + b, pl.ds(0, 16)])
                    new.append(accs[2 * b + 1]
                               + cb * rows[base + b, pl.ds(16, 16)])
                return tuple(new)

            zeros = jnp.zeros((16,), jnp.float32)
            accs = lax.fori_loop(0, _S, body, (zeros,) * (2 * _C))
            for b in range(_C):
                out[b, pl.ds(0, 16)] = jnp.maximum(accs[2 * b], 0.0)
                out[b, pl.ds(16, 16)] = jnp.maximum(accs[2 * b + 1], 0.0)
            base_b = (wid * n_chunks + c) * _C

            @pl.when(base_b + _C <= b_out)
            def _():
                pltpu.async_copy(out_v.at[slot],
                                 out_hbm.at[pl.ds(base_b, _C)], osem[slot])

            @pl.when(base_b + _C > b_out)
            def _():
                pltpu.async_copy(out_v.at[slot], dummy_hbm, osem[slot])

        gathers(0, 0)
        for slot in range(2):
            pltpu.async_copy(out_v.at[slot], dummy_hbm, osem[slot])

        def pair(kk, carry):
            c = 2 * kk
            gathers(c + 1, 1)
            wait_out(0)
            wait_gathers(c, 0)
            compute(c, 0)
            gathers(c + 2, 0)
            wait_out(1)
            wait_gathers(c + 1, 1)
            compute(c + 1, 1)
            return carry

        lax.fori_loop(0, n_chunks // 2, pair, 0)
        wait_gathers(n_chunks, 0)
        for slot in range(2):
            wait_out(slot)

    return k(table, tq, nbr3d, nodeq3d, vslot)[0]


def kernel(nodes, neighbors, feat_table, W, a, U):
    b, s = neighbors.shape
    h = W.shape[1]
    fs = U.shape[0]
    a_ctr, a_nbr = a[0, :h], a[0, h:]
    colsum = jnp.sum(U, axis=0)
    v = jnp.sum(U * colsum[None, :], axis=1)
    vslot = v[1:1 + s].reshape(2, 16)
    a12 = jnp.stack([a_nbr, a_ctr], axis=1)
    th3, tq3 = _premul_table(feat_table, W, a12)
    table = th3.reshape(-1, h)
    tq = tq3.reshape(-1)

    n_chunks = -(-b // (_NW * _C))
    b_pad = _NW * _C * n_chunks
    n_nodes = feat_table.shape[0]
    pad_n = b_pad + 2 * _C - b
    nodes_p = jnp.concatenate(
        [nodes, (jnp.arange(pad_n, dtype=jnp.int32) * 97) % n_nodes])
    nbr_fill = ((jnp.arange(pad_n * _S, dtype=jnp.int32) * 89) % n_nodes
                ).reshape(pad_n, _S)
    nbr_p = jnp.concatenate([neighbors, nbr_fill])
    q = feat_table.shape[0] // 4
    nbr_lo, nbr_hi = nbr_p % q, nbr_p // q
    nbr_rows = nbr_lo * 4 + nbr_hi

    def to3d(x):
        return (x.reshape(-1, _C, _S)
                .transpose(0, 2, 1)
                .reshape(-1, (_C * _S) // 128, 128))

    nbr3d = to3d(nbr_rows)
    nodeq3d = (((nodes_p % q) * 4 + nodes_p // q) * 32 + 1
               ).reshape(-1, 1, _C)
    b_out = b if b % _C == 0 else b_pad
    out = _sc_attention(table, tq, nbr3d, nodeq3d, vslot, n_chunks, b_out)
    return out if b_out == b else out[:b]
